# Initial kernel scaffold; baseline (speedup 1.0000x reference)
import jax
import jax.numpy as jnp
from jax.experimental import pallas as pl


def kernel(x):
    raise NotImplementedError("write your pallas kernel here")


# speedup vs baseline: 2.2624x; 2.2624x over previous
import jax
import jax.numpy as jnp
from jax.experimental import pallas as pl


_GRID = 16


def _copy_block(x_ref, o_ref):
    o_ref[...] = x_ref[...]


def kernel(x):
    b, c, f = x.shape
    blk = b // _GRID
    return pl.pallas_call(
        _copy_block,
        grid=(_GRID,),
        in_specs=[pl.BlockSpec((blk, c, f), lambda i: (i, 0, 0))],
        out_specs=pl.BlockSpec((blk, c, f), lambda i: (i, 0, 0)),
        out_shape=jax.ShapeDtypeStruct((b, c, f), x.dtype),
    )(x)
